# baseline (speedup 1.0000x reference)
import functools

import jax
import jax.numpy as jnp
from jax import lax
from jax.experimental import pallas as pl
from jax.experimental.pallas import tpu as pltpu
from jax.experimental.pallas import tpu_sc as plsc

N = 128
L = 16
NG = N // L
INF = float("inf")


def _bf_body(adj_hbm, col0_hbm, dist_hbm, pred_hbm, neg_hbm,
             adj_loc, prev_ref, newd_ref, dist_loc, pred_loc,
             negacc_ref, negcomb_ref, outv_ref, shared_dist, negshared,
             pub_sem, flo_sem, fhi_sem):
    s = lax.axis_index("s")
    active = s < NG
    vbase = s * L

    H = N // 2

    @pl.when(active)
    def _stage():
        pltpu.sync_copy(adj_hbm.at[:, pl.ds(vbase, L)], adj_loc)
        pltpu.sync_copy(col0_hbm, prev_ref)
        dist_loc[0, :] = prev_ref[pl.ds(vbase, L)]
        pred_loc[0, :] = jnp.zeros((L,), jnp.int32)
        pltpu.sync_copy(prev_ref.at[pl.ds(vbase, L)],
                        shared_dist.at[0, pl.ds(vbase, L)])

    plsc.subcore_barrier()

    @pl.when(active)
    def _prime():
        pltpu.async_copy(shared_dist.at[0, pl.ds(0, H)],
                         prev_ref.at[pl.ds(0, H)], flo_sem)
        pltpu.async_copy(shared_dist.at[0, pl.ds(H, H)],
                         prev_ref.at[pl.ds(H, H)], fhi_sem)

    def step(i, carry):
        @pl.when(active)
        def _compute():
            nch = 4
            per = N // nch
            bests = [jnp.full((L,), INF, jnp.float32) for _ in range(nch)]
            bidxs = [jnp.zeros((L,), jnp.int32) for _ in range(nch)]

            pltpu.make_async_copy(shared_dist.at[(i - 1) % 2, pl.ds(0, H)],
                                  prev_ref.at[pl.ds(0, H)], flo_sem).wait()
            chunks_lo = [prev_ref[pl.ds(16 * c, L)] for c in range(H // L)]
            for cc in range(nch // 2):
                for t in range(per):
                    u = per * cc + t
                    val = chunks_lo[u // L][u % L] + adj_loc[u, :]
                    cond = val < bests[cc]
                    bests[cc] = jnp.minimum(val, bests[cc])
                    bidxs[cc] = jnp.where(cond, u, bidxs[cc])

            pltpu.make_async_copy(shared_dist.at[(i - 1) % 2, pl.ds(H, H)],
                                  prev_ref.at[pl.ds(H, H)], fhi_sem).wait()
            chunks_hi = [prev_ref[pl.ds(H + 16 * c, L)] for c in range(H // L)]
            for cc in range(nch // 2, nch):
                for t in range(per):
                    u = per * cc + t
                    val = chunks_hi[(u - H) // L][u % L] + adj_loc[u, :]
                    cond = val < bests[cc]
                    bests[cc] = jnp.minimum(val, bests[cc])
                    bidxs[cc] = jnp.where(cond, u, bidxs[cc])

            best, bidx = bests[0], bidxs[0]
            for cc in range(1, nch):
                cond = bests[cc] < best
                best = jnp.minimum(bests[cc], best)
                bidx = jnp.where(cond, bidxs[cc], bidx)
            newd_ref[...] = best
            desc = pltpu.async_copy(
                newd_ref, shared_dist.at[i % 2, pl.ds(vbase, L)], pub_sem)
            dist_loc[i, :] = best
            pred_loc[i, :] = bidx
            desc.wait()

        plsc.subcore_barrier()

        @pl.when(active)
        def _issue_fetch():
            pltpu.async_copy(shared_dist.at[i % 2, pl.ds(0, H)],
                             prev_ref.at[pl.ds(0, H)], flo_sem)
            pltpu.async_copy(shared_dist.at[i % 2, pl.ds(H, H)],
                             prev_ref.at[pl.ds(H, H)], fhi_sem)

        return carry

    lax.fori_loop(1, N, step, 0, unroll=2)

    @pl.when(active)
    def _drain():
        pltpu.make_async_copy(shared_dist.at[(N - 1) % 2, pl.ds(0, H)],
                              prev_ref.at[pl.ds(0, H)], flo_sem).wait()
        pltpu.make_async_copy(shared_dist.at[(N - 1) % 2, pl.ds(H, H)],
                              prev_ref.at[pl.ds(H, H)], fhi_sem).wait()

    @pl.when(active)
    def _finish():
        lastv = prev_ref[pl.ds(vbase, L)]

        def nstep(u, c):
            acc, uvec = c
            pv = plsc.load_gather(prev_ref, [uvec])
            val = pv + adj_loc[u, :]
            return acc | (val < lastv), uvec + 1

        acc, _ = lax.fori_loop(
            0, N, nstep, (jnp.zeros((L,), jnp.bool_),
                          jnp.zeros((L,), jnp.int32)))
        negacc_ref[...] = acc.astype(jnp.int32)
        pltpu.sync_copy(negacc_ref, negshared.at[s])
        pltpu.sync_copy(dist_loc, dist_hbm.at[:, pl.ds(vbase, L)])
        pltpu.sync_copy(pred_loc, pred_hbm.at[:, pl.ds(vbase, L)])

    plsc.subcore_barrier()

    @pl.when(s == 0)
    def _combine():
        pltpu.sync_copy(negshared, negcomb_ref)
        flag = jnp.zeros((L,), jnp.int32)
        for g in range(NG):
            flag = flag | negcomb_ref[g, :]
        any_neg = jnp.any(flag != 0)
        outv_ref[...] = jnp.where(any_neg, 1, 0) * jnp.ones((L,), jnp.int32)
        pltpu.sync_copy(outv_ref, neg_hbm)


_bf_call = functools.partial(
    pl.kernel,
    out_type=(
        jax.ShapeDtypeStruct((N, N), jnp.float32),
        jax.ShapeDtypeStruct((N, N), jnp.int32),
        jax.ShapeDtypeStruct((L,), jnp.int32),
    ),
    mesh=plsc.VectorSubcoreMesh(core_axis_name="c", subcore_axis_name="s",
                                num_cores=1),
    compiler_params=pltpu.CompilerParams(use_tc_tiling_on_sc=False,
                                         needs_layout_passes=False),
    scratch_types=(
        pltpu.VMEM((N, L), jnp.float32),
        pltpu.VMEM((N,), jnp.float32),
        pltpu.VMEM((L,), jnp.float32),
        pltpu.VMEM((N, L), jnp.float32),
        pltpu.VMEM((N, L), jnp.int32),
        pltpu.VMEM((L,), jnp.int32),
        pltpu.VMEM((NG, L), jnp.int32),
        pltpu.VMEM((L,), jnp.int32),
        pltpu.VMEM_SHARED((2, N), jnp.float32),
        pltpu.VMEM_SHARED((NG, L), jnp.int32),
        pltpu.SemaphoreType.DMA,
        pltpu.SemaphoreType.DMA,
        pltpu.SemaphoreType.DMA,
    ),
)(_bf_body)


@jax.jit
def kernel(adj_matrix, source_node):
    col0 = jnp.full((N,), INF, jnp.float32).at[source_node].set(0.0)
    dist_it, pred_it, negv = _bf_call(adj_matrix.astype(jnp.float32), col0)
    return dist_it.T, pred_it.T, negv[0] != 0


# speedup vs baseline: 1.1034x; 1.1034x over previous
import functools

import jax
import jax.numpy as jnp
from jax import lax
from jax.experimental import pallas as pl
from jax.experimental.pallas import tpu as pltpu
from jax.experimental.pallas import tpu_sc as plsc

N = 128
L = 16
NG = N // L
H = N // 2
NT = 16
INF = float("inf")


def _bf_body(adj_hbm, col0_hbm, dist_hbm, pred_hbm, neg_hbm,
             adj_loc, fa_ref, gv_ref, gi_ref, newv_ref, newi_ref,
             dist_loc, pred_loc, negacc_ref, negcomb_ref, outv_ref,
             pv_sh, pi_sh, negshared,
             fa_sem, gv_sem, gi_sem, pubv_sem, pubi_sem):
    s = lax.axis_index("s")
    h = s // NG
    p = s % NG
    vbase = p * L
    ubase = h * H

    pltpu.sync_copy(adj_hbm.at[pl.ds(ubase, H), pl.ds(vbase, L)], adj_loc)
    pltpu.sync_copy(col0_hbm.at[pl.ds(vbase, L)],
                    pv_sh.at[0, h, pl.ds(vbase, L)])
    newi_ref[...] = jnp.zeros((L,), jnp.int32)
    pltpu.sync_copy(newi_ref, pi_sh.at[0, h, pl.ds(vbase, L)])

    plsc.subcore_barrier()

    def issue_fetches(par):
        pltpu.async_copy(pv_sh.at[par, :, pl.ds(ubase, H)], fa_ref, fa_sem)

        @pl.when(h == 0)
        def _group():
            pltpu.async_copy(pv_sh.at[par, :, pl.ds(vbase, L)], gv_ref,
                             gv_sem)
            pltpu.async_copy(pi_sh.at[par, :, pl.ds(vbase, L)], gi_ref,
                             gi_sem)

    issue_fetches(0)

    def wait_fa(par):
        pltpu.make_async_copy(pv_sh.at[par, :, pl.ds(ubase, H)], fa_ref,
                              fa_sem).wait()

    def wait_group(par):
        pltpu.make_async_copy(pv_sh.at[par, :, pl.ds(vbase, L)], gv_ref,
                              gv_sem).wait()
        pltpu.make_async_copy(pi_sh.at[par, :, pl.ds(vbase, L)], gi_ref,
                              gi_sem).wait()

    def merged_chunks():
        return [jnp.minimum(fa_ref[0, pl.ds(16 * c, L)],
                            fa_ref[1, pl.ds(16 * c, L)])
                for c in range(H // L)]

    def group_merge():
        gv0 = gv_ref[0, :]
        gv1 = gv_ref[1, :]
        cond = gv1 < gv0
        return jnp.minimum(gv1, gv0), jnp.where(cond, gi_ref[1, :],
                                                gi_ref[0, :])

    def step(i, carry):
        wait_fa((i - 1) % 2)
        chunks = merged_chunks()
        nch = 2
        per = H // nch
        bests = [jnp.full((L,), INF, jnp.float32) for _ in range(nch)]
        bidxs = [jnp.zeros((L,), jnp.int32) for _ in range(nch)]
        for cc in range(nch):
            for t in range(per):
                ul = per * cc + t
                val = chunks[ul // L][ul % L] + adj_loc[ul, :]
                cond = val < bests[cc]
                bests[cc] = jnp.minimum(val, bests[cc])
                bidxs[cc] = jnp.where(cond, ubase + ul, bidxs[cc])
        cond = bests[1] < bests[0]
        best = jnp.minimum(bests[1], bests[0])
        bidx = jnp.where(cond, bidxs[1], bidxs[0])
        newv_ref[...] = best
        newi_ref[...] = bidx
        dv = pltpu.async_copy(newv_ref, pv_sh.at[i % 2, h, pl.ds(vbase, L)],
                              pubv_sem)
        di = pltpu.async_copy(newi_ref, pi_sh.at[i % 2, h, pl.ds(vbase, L)],
                              pubi_sem)

        @pl.when(h == 0)
        def _record():
            wait_group((i - 1) % 2)
            dval, didx = group_merge()
            dist_loc[i - 1, :] = dval
            pred_loc[i - 1, :] = didx

        dv.wait()
        di.wait()
        plsc.subcore_barrier()
        issue_fetches(i % 2)
        return carry

    lax.fori_loop(1, N, step, 0)

    lastpar = (N - 1) % 2
    wait_fa(lastpar)
    lchunks = merged_chunks()

    @pl.when(h == 0)
    def _last_row():
        wait_group(lastpar)
        dval, didx = group_merge()
        dist_loc[N - 1, :] = dval
        pred_loc[N - 1, :] = didx

    @pl.when(h == 1)
    def _late_group():
        pltpu.sync_copy(pv_sh.at[lastpar, :, pl.ds(vbase, L)], gv_ref)

    lastg = jnp.minimum(gv_ref[0, :], gv_ref[1, :])
    acc = jnp.zeros((L,), jnp.bool_)
    for ul in range(H):
        val = lchunks[ul // L][ul % L] + adj_loc[ul, :]
        acc = acc | (val < lastg)
    negacc_ref[...] = acc.astype(jnp.int32)
    pltpu.sync_copy(negacc_ref, negshared.at[s])

    @pl.when(h == 0)
    def _write_out():
        pltpu.sync_copy(dist_loc, dist_hbm.at[:, pl.ds(vbase, L)])
        pltpu.sync_copy(pred_loc, pred_hbm.at[:, pl.ds(vbase, L)])

    plsc.subcore_barrier()

    @pl.when(s == 0)
    def _combine():
        pltpu.sync_copy(negshared, negcomb_ref)
        flag = jnp.zeros((L,), jnp.int32)
        for g in range(NT):
            flag = flag | negcomb_ref[g, :]
        any_neg = jnp.any(flag != 0)
        outv_ref[...] = jnp.where(any_neg, 1, 0) * jnp.ones((L,), jnp.int32)
        pltpu.sync_copy(outv_ref, neg_hbm)


_bf_call = functools.partial(
    pl.kernel,
    out_type=(
        jax.ShapeDtypeStruct((N, N), jnp.float32),
        jax.ShapeDtypeStruct((N, N), jnp.int32),
        jax.ShapeDtypeStruct((L,), jnp.int32),
    ),
    mesh=plsc.VectorSubcoreMesh(core_axis_name="c", subcore_axis_name="s",
                                num_cores=1),
    compiler_params=pltpu.CompilerParams(use_tc_tiling_on_sc=False,
                                         needs_layout_passes=False),
    scratch_types=(
        pltpu.VMEM((H, L), jnp.float32),
        pltpu.VMEM((2, H), jnp.float32),
        pltpu.VMEM((2, L), jnp.float32),
        pltpu.VMEM((2, L), jnp.int32),
        pltpu.VMEM((L,), jnp.float32),
        pltpu.VMEM((L,), jnp.int32),
        pltpu.VMEM((N, L), jnp.float32),
        pltpu.VMEM((N, L), jnp.int32),
        pltpu.VMEM((L,), jnp.int32),
        pltpu.VMEM((NT, L), jnp.int32),
        pltpu.VMEM((L,), jnp.int32),
        pltpu.VMEM_SHARED((2, 2, N), jnp.float32),
        pltpu.VMEM_SHARED((2, 2, N), jnp.int32),
        pltpu.VMEM_SHARED((NT, L), jnp.int32),
        pltpu.SemaphoreType.DMA,
        pltpu.SemaphoreType.DMA,
        pltpu.SemaphoreType.DMA,
        pltpu.SemaphoreType.DMA,
        pltpu.SemaphoreType.DMA,
    ),
)(_bf_body)


@jax.jit
def kernel(adj_matrix, source_node):
    col0 = jnp.full((N,), INF, jnp.float32).at[source_node].set(0.0)
    dist_it, pred_it, negv = _bf_call(adj_matrix.astype(jnp.float32), col0)
    return dist_it.T, pred_it.T, negv[0] != 0


# speedup vs baseline: 1.1053x; 1.0018x over previous
import functools

import jax
import jax.numpy as jnp
from jax import lax
from jax.experimental import pallas as pl
from jax.experimental.pallas import tpu as pltpu
from jax.experimental.pallas import tpu_sc as plsc

N = 128
L = 16
NG = N // L
H = N // 2
NT = 16
INF = float("inf")


def _bf_body(adj_hbm, col0_hbm, dist_hbm, pred_hbm, neg_hbm,
             adj_loc, fa_ref, gvi_ref, newvi_ref,
             dist_loc, pred_loc, negacc_ref, negcomb_ref, outv_ref,
             pvi_sh, negshared,
             fa_sem, g_sem, pub_sem):
    s = lax.axis_index("s")
    h = s // NG
    p = s % NG
    vbase = p * L
    ubase = h * H

    pltpu.sync_copy(adj_hbm.at[pl.ds(ubase, H), pl.ds(vbase, L)], adj_loc)
    pltpu.sync_copy(col0_hbm.at[pl.ds(vbase, L)],
                    pvi_sh.at[0, h, 0, pl.ds(vbase, L)])
    newvi_ref[0, :] = jnp.zeros((L,), jnp.float32)
    pltpu.sync_copy(newvi_ref.at[0], pvi_sh.at[0, h, 1, pl.ds(vbase, L)])

    plsc.subcore_barrier()

    def issue_fetches(par):
        pltpu.async_copy(pvi_sh.at[par, :, 0, pl.ds(ubase, H)], fa_ref,
                         fa_sem)

        @pl.when(h == 0)
        def _group():
            pltpu.async_copy(pvi_sh.at[par, :, :, pl.ds(vbase, L)], gvi_ref,
                             g_sem)

    issue_fetches(0)

    def wait_fa(par):
        pltpu.make_async_copy(pvi_sh.at[par, :, 0, pl.ds(ubase, H)], fa_ref,
                              fa_sem).wait()

    def wait_group(par):
        pltpu.make_async_copy(pvi_sh.at[par, :, :, pl.ds(vbase, L)], gvi_ref,
                              g_sem).wait()

    def merged_chunks():
        return [jnp.minimum(fa_ref[0, pl.ds(16 * c, L)],
                            fa_ref[1, pl.ds(16 * c, L)])
                for c in range(H // L)]

    def group_merge():
        gv0 = gvi_ref[0, 0, :]
        gv1 = gvi_ref[1, 0, :]
        gi0 = plsc.bitcast(gvi_ref[0, 1, :], jnp.int32)
        gi1 = plsc.bitcast(gvi_ref[1, 1, :], jnp.int32)
        cond = gv1 < gv0
        return jnp.minimum(gv1, gv0), jnp.where(cond, gi1, gi0)

    def step(i, carry):
        wait_fa((i - 1) % 2)
        chunks = merged_chunks()
        nch = 2
        per = H // nch
        bests = [jnp.full((L,), INF, jnp.float32) for _ in range(nch)]
        bidxs = [jnp.zeros((L,), jnp.int32) for _ in range(nch)]
        for cc in range(nch):
            for t in range(per):
                ul = per * cc + t
                val = chunks[ul // L][ul % L] + adj_loc[ul, :]
                cond = val < bests[cc]
                bests[cc] = jnp.minimum(val, bests[cc])
                bidxs[cc] = jnp.where(cond, ubase + ul, bidxs[cc])
        cond = bests[1] < bests[0]
        best = jnp.minimum(bests[1], bests[0])
        bidx = jnp.where(cond, bidxs[1], bidxs[0])
        newvi_ref[0, :] = best
        newvi_ref[1, :] = plsc.bitcast(bidx, jnp.float32)
        dv = pltpu.async_copy(newvi_ref,
                              pvi_sh.at[i % 2, h, :, pl.ds(vbase, L)],
                              pub_sem)

        @pl.when(h == 0)
        def _record():
            wait_group((i - 1) % 2)
            dval, didx = group_merge()
            dist_loc[i - 1, :] = dval
            pred_loc[i - 1, :] = didx

        dv.wait()
        plsc.subcore_barrier()
        issue_fetches(i % 2)
        return carry

    lax.fori_loop(1, N, step, 0)

    lastpar = (N - 1) % 2
    wait_fa(lastpar)
    lchunks = merged_chunks()

    @pl.when(h == 0)
    def _last_row():
        wait_group(lastpar)
        dval, didx = group_merge()
        dist_loc[N - 1, :] = dval
        pred_loc[N - 1, :] = didx

    @pl.when(h == 1)
    def _late_group():
        pltpu.sync_copy(pvi_sh.at[lastpar, :, :, pl.ds(vbase, L)], gvi_ref)

    lastg = jnp.minimum(gvi_ref[0, 0, :], gvi_ref[1, 0, :])
    acc = jnp.zeros((L,), jnp.bool_)
    for ul in range(H):
        val = lchunks[ul // L][ul % L] + adj_loc[ul, :]
        acc = acc | (val < lastg)
    negacc_ref[...] = acc.astype(jnp.int32)
    pltpu.sync_copy(negacc_ref, negshared.at[s])

    @pl.when(h == 0)
    def _write_out():
        pltpu.sync_copy(dist_loc, dist_hbm.at[:, pl.ds(vbase, L)])
        pltpu.sync_copy(pred_loc, pred_hbm.at[:, pl.ds(vbase, L)])

    plsc.subcore_barrier()

    @pl.when(s == 0)
    def _combine():
        pltpu.sync_copy(negshared, negcomb_ref)
        flag = jnp.zeros((L,), jnp.int32)
        for g in range(NT):
            flag = flag | negcomb_ref[g, :]
        any_neg = jnp.any(flag != 0)
        outv_ref[...] = jnp.where(any_neg, 1, 0) * jnp.ones((L,), jnp.int32)
        pltpu.sync_copy(outv_ref, neg_hbm)


_bf_call = functools.partial(
    pl.kernel,
    out_type=(
        jax.ShapeDtypeStruct((N, N), jnp.float32),
        jax.ShapeDtypeStruct((N, N), jnp.int32),
        jax.ShapeDtypeStruct((L,), jnp.int32),
    ),
    mesh=plsc.VectorSubcoreMesh(core_axis_name="c", subcore_axis_name="s",
                                num_cores=1),
    compiler_params=pltpu.CompilerParams(use_tc_tiling_on_sc=False,
                                         needs_layout_passes=False),
    scratch_types=(
        pltpu.VMEM((H, L), jnp.float32),
        pltpu.VMEM((2, H), jnp.float32),
        pltpu.VMEM((2, 2, L), jnp.float32),
        pltpu.VMEM((2, L), jnp.float32),
        pltpu.VMEM((N, L), jnp.float32),
        pltpu.VMEM((N, L), jnp.int32),
        pltpu.VMEM((L,), jnp.int32),
        pltpu.VMEM((NT, L), jnp.int32),
        pltpu.VMEM((L,), jnp.int32),
        pltpu.VMEM_SHARED((2, 2, 2, N), jnp.float32),
        pltpu.VMEM_SHARED((NT, L), jnp.int32),
        pltpu.SemaphoreType.DMA,
        pltpu.SemaphoreType.DMA,
        pltpu.SemaphoreType.DMA,
    ),
)(_bf_body)


@jax.jit
def kernel(adj_matrix, source_node):
    col0 = jnp.full((N,), INF, jnp.float32).at[source_node].set(0.0)
    dist_it, pred_it, negv = _bf_call(adj_matrix.astype(jnp.float32), col0)
    return dist_it.T, pred_it.T, negv[0] != 0


# speedup vs baseline: 1.1376x; 1.0292x over previous
import functools

import jax
import jax.numpy as jnp
from jax import lax
from jax.experimental import pallas as pl
from jax.experimental.pallas import tpu as pltpu
from jax.experimental.pallas import tpu_sc as plsc

N = 128
L = 16
NG = N // L
H = N // 2
NT = 16
INF = float("inf")


def _bf_body(adj_hbm, col0_hbm, dist_hbm, pred_hbm, neg_hbm,
             adj_loc, fa_ref, gvi_ref, newvi_ref,
             dist_loc, pred_loc, negacc_ref, negcomb_ref, outv_ref,
             pvi_sh, negshared,
             fa_sem, fb_sem, g_sem, pub_sem):
    s = lax.axis_index("s")
    h = s // NG
    p = s % NG
    vbase = p * L
    ubase = h * H

    pltpu.sync_copy(adj_hbm.at[pl.ds(ubase, H), pl.ds(vbase, L)], adj_loc)
    pltpu.sync_copy(col0_hbm.at[pl.ds(vbase, L)],
                    pvi_sh.at[0, h, 0, pl.ds(vbase, L)])
    newvi_ref[0, :] = jnp.zeros((L,), jnp.float32)
    pltpu.sync_copy(newvi_ref.at[0], pvi_sh.at[0, h, 1, pl.ds(vbase, L)])

    plsc.subcore_barrier()

    Q = H // 2

    def issue_fetches(par):
        pltpu.async_copy(pvi_sh.at[par, :, 0, pl.ds(ubase, Q)],
                         fa_ref.at[:, pl.ds(0, Q)], fa_sem)
        pltpu.async_copy(pvi_sh.at[par, :, 0, pl.ds(ubase + Q, Q)],
                         fa_ref.at[:, pl.ds(Q, Q)], fb_sem)

        @pl.when(h == 0)
        def _group():
            pltpu.async_copy(pvi_sh.at[par, :, :, pl.ds(vbase, L)], gvi_ref,
                             g_sem)

    issue_fetches(0)

    def wait_fa(par):
        pltpu.make_async_copy(pvi_sh.at[par, :, 0, pl.ds(ubase, Q)],
                              fa_ref.at[:, pl.ds(0, Q)], fa_sem).wait()

    def wait_fb(par):
        pltpu.make_async_copy(pvi_sh.at[par, :, 0, pl.ds(ubase + Q, Q)],
                              fa_ref.at[:, pl.ds(Q, Q)], fb_sem).wait()

    def wait_group(par):
        pltpu.make_async_copy(pvi_sh.at[par, :, :, pl.ds(vbase, L)], gvi_ref,
                              g_sem).wait()

    def merged_chunks(lo, hi):
        return [jnp.minimum(fa_ref[0, pl.ds(16 * c, L)],
                            fa_ref[1, pl.ds(16 * c, L)])
                for c in range(lo, hi)]

    def group_merge():
        gv0 = gvi_ref[0, 0, :]
        gv1 = gvi_ref[1, 0, :]
        gi0 = plsc.bitcast(gvi_ref[0, 1, :], jnp.int32)
        gi1 = plsc.bitcast(gvi_ref[1, 1, :], jnp.int32)
        cond = gv1 < gv0
        return jnp.minimum(gv1, gv0), jnp.where(cond, gi1, gi0)

    def step(i, carry):
        nch = 2
        per = H // nch
        bests = [jnp.full((L,), INF, jnp.float32) for _ in range(nch)]
        bidxs = [jnp.zeros((L,), jnp.int32) for _ in range(nch)]
        wait_fa((i - 1) % 2)
        chunks = merged_chunks(0, 2)
        for t in range(per):
            val = chunks[t // L][t % L] + adj_loc[t, :]
            cond = val < bests[0]
            bests[0] = jnp.minimum(val, bests[0])
            bidxs[0] = jnp.where(cond, ubase + t, bidxs[0])
        wait_fb((i - 1) % 2)
        chunks_hi = merged_chunks(2, 4)
        for t in range(per):
            ul = per + t
            val = chunks_hi[t // L][t % L] + adj_loc[ul, :]
            cond = val < bests[1]
            bests[1] = jnp.minimum(val, bests[1])
            bidxs[1] = jnp.where(cond, ubase + ul, bidxs[1])
        cond = bests[1] < bests[0]
        best = jnp.minimum(bests[1], bests[0])
        bidx = jnp.where(cond, bidxs[1], bidxs[0])
        newvi_ref[0, :] = best
        newvi_ref[1, :] = plsc.bitcast(bidx, jnp.float32)
        dv = pltpu.async_copy(newvi_ref,
                              pvi_sh.at[i % 2, h, :, pl.ds(vbase, L)],
                              pub_sem)

        @pl.when(h == 0)
        def _record():
            wait_group((i - 1) % 2)
            dval, didx = group_merge()
            dist_loc[i - 1, :] = dval
            pred_loc[i - 1, :] = didx

        dv.wait()
        plsc.subcore_barrier()
        issue_fetches(i % 2)
        return carry

    lax.fori_loop(1, N, step, 0)

    lastpar = (N - 1) % 2
    wait_fa(lastpar)
    wait_fb(lastpar)
    lchunks = merged_chunks(0, 4)

    @pl.when(h == 0)
    def _last_row():
        wait_group(lastpar)
        dval, didx = group_merge()
        dist_loc[N - 1, :] = dval
        pred_loc[N - 1, :] = didx

    @pl.when(h == 1)
    def _late_group():
        pltpu.sync_copy(pvi_sh.at[lastpar, :, :, pl.ds(vbase, L)], gvi_ref)

    lastg = jnp.minimum(gvi_ref[0, 0, :], gvi_ref[1, 0, :])
    acc = jnp.zeros((L,), jnp.bool_)
    for ul in range(H):
        val = lchunks[ul // L][ul % L] + adj_loc[ul, :]
        acc = acc | (val < lastg)
    negacc_ref[...] = acc.astype(jnp.int32)
    pltpu.sync_copy(negacc_ref, negshared.at[s])

    @pl.when(h == 0)
    def _write_out():
        pltpu.sync_copy(dist_loc, dist_hbm.at[:, pl.ds(vbase, L)])
        pltpu.sync_copy(pred_loc, pred_hbm.at[:, pl.ds(vbase, L)])

    plsc.subcore_barrier()

    @pl.when(s == 0)
    def _combine():
        pltpu.sync_copy(negshared, negcomb_ref)
        flag = jnp.zeros((L,), jnp.int32)
        for g in range(NT):
            flag = flag | negcomb_ref[g, :]
        any_neg = jnp.any(flag != 0)
        outv_ref[...] = jnp.where(any_neg, 1, 0) * jnp.ones((L,), jnp.int32)
        pltpu.sync_copy(outv_ref, neg_hbm)


_bf_call = functools.partial(
    pl.kernel,
    out_type=(
        jax.ShapeDtypeStruct((N, N), jnp.float32),
        jax.ShapeDtypeStruct((N, N), jnp.int32),
        jax.ShapeDtypeStruct((L,), jnp.int32),
    ),
    mesh=plsc.VectorSubcoreMesh(core_axis_name="c", subcore_axis_name="s",
                                num_cores=1),
    compiler_params=pltpu.CompilerParams(use_tc_tiling_on_sc=False,
                                         needs_layout_passes=False),
    scratch_types=(
        pltpu.VMEM((H, L), jnp.float32),
        pltpu.VMEM((2, H), jnp.float32),
        pltpu.VMEM((2, 2, L), jnp.float32),
        pltpu.VMEM((2, L), jnp.float32),
        pltpu.VMEM((N, L), jnp.float32),
        pltpu.VMEM((N, L), jnp.int32),
        pltpu.VMEM((L,), jnp.int32),
        pltpu.VMEM((NT, L), jnp.int32),
        pltpu.VMEM((L,), jnp.int32),
        pltpu.VMEM_SHARED((2, 2, 2, N), jnp.float32),
        pltpu.VMEM_SHARED((NT, L), jnp.int32),
        pltpu.SemaphoreType.DMA,
        pltpu.SemaphoreType.DMA,
        pltpu.SemaphoreType.DMA,
        pltpu.SemaphoreType.DMA,
    ),
)(_bf_body)


@jax.jit
def kernel(adj_matrix, source_node):
    col0 = jnp.full((N,), INF, jnp.float32).at[source_node].set(0.0)
    dist_it, pred_it, negv = _bf_call(adj_matrix.astype(jnp.float32), col0)
    return dist_it.T, pred_it.T, negv[0] != 0
